# Initial kernel scaffold; baseline (speedup 1.0000x reference)
import jax
import jax.numpy as jnp
from jax.experimental import pallas as pl


def kernel(node_input, node_attr_input, node_attr_output, edge_src, edge_dst, edge_attr, edge_scalar_attr, W_lin1, fc_w0, fc_w1, fc_w2, W_lin2):
    raise NotImplementedError("write your pallas kernel here")


# speedup vs baseline: 1.2102x; 1.2102x over previous
import functools
import math

import jax
import jax.numpy as jnp
from jax import lax
from jax.experimental import pallas as pl
from jax.experimental.pallas import tpu as pltpu
from jax.experimental.pallas import tpu_sc as plsc

_N = 10000
_NPAD = 10240
_E = 320000
_D = 128
_NUM_NEIGHBORS = 32.0
_SILU_NORM = 1.679177

_NC = 2
_NSUB = 16
_NW = _NC * _NSUB
_CHUNK = 128
_CPW = 80
_IGRP = 16
_EPAD = _NW * _CPW * _CHUNK
_ROWS_PER_TILE = _NPAD // _NSUB

_BE = 2048


def _mlp_body(es_ref, ea_ref, w0_ref, w1_ref, w2_ref, out_ref):
    h = jnp.dot(es_ref[...], w0_ref[...], preferred_element_type=jnp.float32)
    h = jax.nn.silu(h) * _SILU_NORM
    h = jnp.dot(h, w1_ref[...], preferred_element_type=jnp.float32)
    h = jax.nn.silu(h) * _SILU_NORM
    w = jnp.dot(h, w2_ref[...], preferred_element_type=jnp.float32)
    out_ref[...] = w * ea_ref[...]


def _lin1_body(x_ref, attr_ref, w_ref, o_ref):
    o_ref[...] = (
        jnp.dot(x_ref[...], w_ref[...], preferred_element_type=jnp.float32)
        * attr_ref[...]
    )


def _lin2_body(p_ref, attr_ref, w_ref, o_ref):
    s = p_ref[0, :_N] + p_ref[1, :_N]
    o_ref[...] = (
        jnp.dot(s, w_ref[...], preferred_element_type=jnp.float32) * attr_ref[...]
    )


def _sc_body(x_hbm, wmod_hbm, src_hbm, dst_hbm, zeros_hbm, out_hbm,
             src_v, dst_v, rows_v, wm_v, acc_sh, sem):
    c = lax.axis_index("c")
    s = lax.axis_index("s")
    wid = s * _NC + c

    pltpu.sync_copy(
        zeros_hbm.at[pl.ds(s * _ROWS_PER_TILE, _ROWS_PER_TILE)],
        acc_sh.at[pl.ds(s * _ROWS_PER_TILE, _ROWS_PER_TILE)],
    )
    plsc.subcore_barrier()

    base = wid * _CPW

    def group_body(g, carry):
        gbase = base + g * _IGRP
        pltpu.sync_copy(src_hbm.at[pl.ds(gbase, _IGRP)], src_v)
        pltpu.sync_copy(dst_hbm.at[pl.ds(gbase, _IGRP)], dst_v)

        def chunk_body(j, carry1):
            pltpu.async_copy(x_hbm.at[src_v.at[j]], rows_v, sem).wait()
            pltpu.sync_copy(
                wmod_hbm.at[pl.ds((gbase + j) * _CHUNK, _CHUNK)], wm_v)

            def mul_row(i, carry2):
                for k in range(_D // 16):
                    sl = pl.ds(k * 16, 16)
                    rows_v[i, sl] = rows_v[i, sl] * wm_v[i, sl]
                return carry2

            lax.fori_loop(0, _CHUNK, mul_row, 0, unroll=2)

            pltpu.sync_copy(rows_v, acc_sh.at[dst_v.at[j]], add=True)
            return carry1

        lax.fori_loop(0, _IGRP, chunk_body, 0)
        return carry

    lax.fori_loop(0, _CPW // _IGRP, group_body, 0)
    plsc.subcore_barrier()

    pltpu.sync_copy(
        acc_sh.at[pl.ds(s * _ROWS_PER_TILE, _ROWS_PER_TILE)],
        out_hbm.at[c, pl.ds(s * _ROWS_PER_TILE, _ROWS_PER_TILE)],
    )


_sc_scatter = pl.kernel(
    _sc_body,
    out_type=jax.ShapeDtypeStruct((_NC, _NPAD, _D), jnp.float32),
    mesh=plsc.VectorSubcoreMesh(
        core_axis_name="c", subcore_axis_name="s",
        num_cores=_NC, num_subcores=_NSUB),
    scratch_types=[
        pltpu.VMEM((_IGRP, _CHUNK), jnp.int32),
        pltpu.VMEM((_IGRP, _CHUNK), jnp.int32),
        pltpu.VMEM((_CHUNK, _D), jnp.float32),
        pltpu.VMEM((_CHUNK, _D), jnp.float32),
        pltpu.VMEM_SHARED((_NPAD, _D), jnp.float32),
        pltpu.SemaphoreType.DMA,
    ],
)


@jax.jit
def kernel(node_input, node_attr_input, node_attr_output, edge_src, edge_dst,
           edge_attr, edge_scalar_attr, W_lin1, fc_w0, fc_w1, fc_w2, W_lin2):
    w0s = fc_w0 * (1.0 / math.sqrt(fc_w0.shape[0]))
    w1s = fc_w1 * (1.0 / math.sqrt(fc_w1.shape[0]))
    w2s = fc_w2 * (1.0 / math.sqrt(fc_w2.shape[0]))
    w_lin1s = W_lin1 * (1.0 / math.sqrt(_D))
    w_lin2s = W_lin2 * (1.0 / (math.sqrt(_D) * math.sqrt(_NUM_NEIGHBORS)))

    pad = _EPAD - _E
    es_p = jnp.pad(edge_scalar_attr, ((0, pad), (0, 0)))
    ea_p = jnp.pad(edge_attr, ((0, pad), (0, 0)))
    src_p = jnp.pad(edge_src.astype(jnp.int32), (0, pad)).reshape(
        _NW * _CPW, _CHUNK)
    dst_p = jnp.pad(edge_dst.astype(jnp.int32), (0, pad)).reshape(
        _NW * _CPW, _CHUNK)
    zeros = jnp.zeros((_NPAD, _D), jnp.float32)

    wmod = pl.pallas_call(
        _mlp_body,
        grid=(_EPAD // _BE,),
        in_specs=[
            pl.BlockSpec((_BE, 16), lambda i: (i, 0)),
            pl.BlockSpec((_BE, 1), lambda i: (i, 0)),
            pl.BlockSpec((16, 64), lambda i: (0, 0)),
            pl.BlockSpec((64, 64), lambda i: (0, 0)),
            pl.BlockSpec((64, _D), lambda i: (0, 0)),
        ],
        out_specs=pl.BlockSpec((_BE, _D), lambda i: (i, 0)),
        out_shape=jax.ShapeDtypeStruct((_EPAD, _D), jnp.float32),
    )(es_p, ea_p, w0s, w1s, w2s)

    x = pl.pallas_call(
        _lin1_body,
        out_shape=jax.ShapeDtypeStruct((_N, _D), jnp.float32),
    )(node_input, node_attr_input, w_lin1s)

    parts = _sc_scatter(x, wmod, src_p, dst_p, zeros)

    return pl.pallas_call(
        _lin2_body,
        out_shape=jax.ShapeDtypeStruct((_N, _D), jnp.float32),
    )(parts, node_attr_output, w_lin2s)
